# Initial kernel scaffold; baseline (speedup 1.0000x reference)
import jax
import jax.numpy as jnp
from jax.experimental import pallas as pl


def kernel(x, edge_index, edge_weight, W1, b1, W2, b2, W3, b3):
    raise NotImplementedError("write your pallas kernel here")


# speedup vs baseline: 7.9116x; 7.9116x over previous
import functools

import jax
import jax.numpy as jnp
from jax import lax
from jax.experimental import pallas as pl
from jax.experimental.pallas import tpu as pltpu
from jax.experimental.pallas import tpu_sc as plsc

N = 10000
NP = 10240
D = 128
E = 320000
EPAD = 327680
ECH = EPAD // 128
NC = 2
NS = 16
NW = NC * NS
RPT = NP // NS
CPT = ECH // NW
L = 16
BM = 256


def _rsqrt16(d):
    i = lax.bitcast_convert_type(d, jnp.int32)
    i = jnp.int32(0x5F3759DF) - (i >> 1)
    r = lax.bitcast_convert_type(i, jnp.float32)
    for _ in range(3):
        r = r * (1.5 - 0.5 * d * r * r)
    return r


def _bcast_lane(v, j):
    return jnp.take_along_axis(
        v, jnp.full((L,), j, jnp.int32), axis=0, mode="promise_in_bounds"
    )


def _scale_scatter(xw_hbm, acc_sp, bufA, bufC, bufD, rows_v, sem, dis_sp=None,
                   dis_chunk=None):
    @pl.loop(0, CPT)
    def _chunk(i):
        gat = pltpu.async_copy(xw_hbm.at[bufA.at[i]], rows_v, sem)
        if dis_sp is not None:
            pltpu.sync_copy(dis_sp.at[bufA.at[i]], dis_chunk)

            @pl.loop(0, 128 // L)
            def _nrm(g):
                g16 = pl.multiple_of(g * L, L)
                sl = pl.ds(g16, L)
                bufD[i, sl] = bufD[i, sl] * dis_chunk[sl]

        gat.wait()

        @pl.loop(0, 128 // L)
        def _grp(g):
            g16 = pl.multiple_of(g * L, L)
            nv = bufD[i, pl.ds(g16, L)]
            for j in range(L):
                nb = _bcast_lane(nv, j)
                e = g16 + j
                for d in range(D // L):
                    sl = pl.ds(d * L, L)
                    rows_v[e, sl] = rows_v[e, sl] * nb

        pltpu.sync_copy(rows_v, acc_sp.at[bufC.at[i]], add=True)


def _sc1_body(row_hbm, col_hbm, ew_hbm, xw_hbm, z2d_hbm, z1d_hbm,
              part_hbm, dis_hbm, normp_hbm,
              acc_sp, deg_sp, dis_sp,
              bufA, bufB, bufC, rows_v, small, dis_chunk, sem):
    c = lax.axis_index("c")
    s = lax.axis_index("s")
    w = s * NC + c

    pltpu.sync_copy(z2d_hbm, acc_sp.at[pl.ds(s * RPT, RPT)])
    pltpu.sync_copy(z1d_hbm, deg_sp.at[pl.ds(s * RPT, RPT)])
    plsc.subcore_barrier()

    for half in range(2):
        base = s * 2 * CPT + half * CPT

        pltpu.sync_copy(col_hbm.at[pl.ds(base, CPT)], bufA)
        pltpu.sync_copy(ew_hbm.at[pl.ds(base, CPT)], bufB)

        @pl.loop(0, CPT)
        def _deg(i):
            pltpu.sync_copy(bufB.at[i], deg_sp.at[bufA.at[i]], add=True)

    plsc.subcore_barrier()

    for piece in range(RPT // 128):
        pbase = s * RPT + piece * 128
        pltpu.sync_copy(deg_sp.at[pl.ds(pbase, 128)], small)

        @pl.loop(0, 128 // L)
        def _dis(g):
            g16 = pl.multiple_of(g * L, L)
            small[pl.ds(g16, L)] = _rsqrt16(small[pl.ds(g16, L)] + 1.0)

        pltpu.sync_copy(small, dis_sp.at[pl.ds(pbase, 128)])

        @pl.when(c == 0)
        def _():
            pltpu.sync_copy(small, dis_hbm.at[pl.ds(pbase, 128)])

    plsc.subcore_barrier()

    ebase = w * CPT
    pltpu.sync_copy(row_hbm.at[pl.ds(ebase, CPT)], bufA)
    pltpu.sync_copy(ew_hbm.at[pl.ds(ebase, CPT)], bufB)
    pltpu.sync_copy(col_hbm.at[pl.ds(ebase, CPT)], bufC)

    _scale_scatter(xw_hbm, acc_sp, bufA, bufC, bufB, rows_v, sem,
                   dis_sp=dis_sp, dis_chunk=dis_chunk)
    pltpu.sync_copy(bufB, normp_hbm.at[pl.ds(ebase, CPT)])
    plsc.subcore_barrier()

    pltpu.sync_copy(acc_sp.at[pl.ds(s * RPT, RPT)],
                    part_hbm.at[c, pl.ds(s * RPT, RPT)])


def _sc2_body(row_hbm, col_hbm, np_hbm, xw_hbm, z2d_hbm,
              part_hbm,
              acc_sp, bufA, bufC, bufD, rows_v, sem):
    c = lax.axis_index("c")
    s = lax.axis_index("s")
    w = s * NC + c

    pltpu.sync_copy(z2d_hbm, acc_sp.at[pl.ds(s * RPT, RPT)])

    ebase = w * CPT
    pltpu.sync_copy(row_hbm.at[pl.ds(ebase, CPT)], bufA)
    pltpu.sync_copy(col_hbm.at[pl.ds(ebase, CPT)], bufC)
    pltpu.sync_copy(np_hbm.at[pl.ds(ebase, CPT)], bufD)
    plsc.subcore_barrier()

    _scale_scatter(xw_hbm, acc_sp, bufA, bufC, bufD, rows_v, sem)
    plsc.subcore_barrier()

    pltpu.sync_copy(acc_sp.at[pl.ds(s * RPT, RPT)],
                    part_hbm.at[c, pl.ds(s * RPT, RPT)])


@functools.lru_cache(maxsize=None)
def _sc1():
    mesh = plsc.VectorSubcoreMesh(core_axis_name="c", subcore_axis_name="s")
    return pl.kernel(
        _sc1_body,
        out_type=[
            jax.ShapeDtypeStruct((NC, NP, D), jnp.float32),
            jax.ShapeDtypeStruct((NP,), jnp.float32),
            jax.ShapeDtypeStruct((ECH, 128), jnp.float32),
        ],
        mesh=mesh,
        compiler_params=pltpu.CompilerParams(needs_layout_passes=False),
        scratch_types=[
            pltpu.VMEM_SHARED((NP, D), jnp.float32),
            pltpu.VMEM_SHARED((NP,), jnp.float32),
            pltpu.VMEM_SHARED((NP,), jnp.float32),
            pltpu.VMEM((CPT, 128), jnp.int32),
            pltpu.VMEM((CPT, 128), jnp.float32),
            pltpu.VMEM((CPT, 128), jnp.int32),
            pltpu.VMEM((128, D), jnp.float32),
            pltpu.VMEM((128,), jnp.float32),
            pltpu.VMEM((128,), jnp.float32),
            pltpu.SemaphoreType.DMA,
        ],
    )


@functools.lru_cache(maxsize=None)
def _sc2():
    mesh = plsc.VectorSubcoreMesh(core_axis_name="c", subcore_axis_name="s")
    return pl.kernel(
        _sc2_body,
        out_type=[jax.ShapeDtypeStruct((NC, NP, D), jnp.float32)],
        mesh=mesh,
        compiler_params=pltpu.CompilerParams(needs_layout_passes=False),
        scratch_types=[
            pltpu.VMEM_SHARED((NP, D), jnp.float32),
            pltpu.VMEM((CPT, 128), jnp.int32),
            pltpu.VMEM((CPT, 128), jnp.int32),
            pltpu.VMEM((CPT, 128), jnp.float32),
            pltpu.VMEM((128, D), jnp.float32),
            pltpu.SemaphoreType.DMA,
        ],
    )


def _mm_body(x_ref, w_ref, o_ref):
    o_ref[...] = jnp.dot(x_ref[...], w_ref[...],
                         preferred_element_type=jnp.float32)


def _m2_body(p0_ref, p1_ref, xw_ref, dis_ref, b_ref, w_ref, o_ref):
    dis = dis_ref[...]
    pre = (p0_ref[...] + p1_ref[...] + dis * xw_ref[...]) * dis + b_ref[...]
    h = jnp.maximum(pre, 0.0)
    o_ref[...] = jnp.dot(h, w_ref[...], preferred_element_type=jnp.float32)


def _m3_body(p0_ref, p1_ref, xw_ref, dis_ref, b_ref, w_ref, b3_ref, o_ref):
    dis = dis_ref[...]
    pre = (p0_ref[...] + p1_ref[...] + dis * xw_ref[...]) * dis + b_ref[...]
    h = jnp.maximum(pre, 0.0)
    o_ref[...] = jnp.tanh(
        jnp.dot(h, w_ref[...], preferred_element_type=jnp.float32)
        + b3_ref[...]
    )


_row_spec = pl.BlockSpec((BM, D), lambda i: (i, 0))
_w_spec = pl.BlockSpec((D, D), lambda i: (0, 0))
_b_spec = pl.BlockSpec((1, D), lambda i: (0, 0))
_dis_spec = pl.BlockSpec((BM, 1), lambda i: (i, 0))
_out_sds = jax.ShapeDtypeStruct((NP, D), jnp.float32)


def _mm(x, w):
    return pl.pallas_call(
        _mm_body,
        grid=(NP // BM,),
        in_specs=[_row_spec, _w_spec],
        out_specs=_row_spec,
        out_shape=_out_sds,
    )(x, w)


def _m2(p0, p1, xw, dis2d, b, w):
    return pl.pallas_call(
        _m2_body,
        grid=(NP // BM,),
        in_specs=[_row_spec, _row_spec, _row_spec, _dis_spec, _b_spec, _w_spec],
        out_specs=_row_spec,
        out_shape=_out_sds,
    )(p0, p1, xw, dis2d, b, w)


def _m3(p0, p1, xw, dis2d, b, w, b3):
    return pl.pallas_call(
        _m3_body,
        grid=(NP // BM,),
        in_specs=[_row_spec, _row_spec, _row_spec, _dis_spec, _b_spec, _w_spec,
                  _b_spec],
        out_specs=_row_spec,
        out_shape=_out_sds,
    )(p0, p1, xw, dis2d, b, w, b3)


def kernel(x, edge_index, edge_weight, W1, b1, W2, b2, W3, b3):
    row = edge_index[0].astype(jnp.int32)
    col = edge_index[1].astype(jnp.int32)
    ew = edge_weight.astype(jnp.float32)

    pad = EPAD - E
    row_p = jnp.concatenate([row, jnp.zeros((pad,), jnp.int32)]).reshape(ECH, 128)
    col_p = jnp.concatenate([col, jnp.full((pad,), N, jnp.int32)]).reshape(ECH, 128)
    ew_p = jnp.concatenate([ew, jnp.zeros((pad,), jnp.float32)]).reshape(ECH, 128)
    x_p = jnp.pad(x, ((0, NP - N), (0, 0)))
    z2d = jnp.zeros((RPT, D), jnp.float32)
    z1d = jnp.zeros((RPT,), jnp.float32)

    xw1 = _mm(x_p, W1.astype(jnp.float32))
    part1, dis, normp = _sc1()(row_p, col_p, ew_p, xw1, z2d, z1d)
    dis2d = dis[:, None]
    xw2 = _m2(part1[0], part1[1], xw1, dis2d, b1.reshape(1, D), W2)
    (part2,) = (_sc2()(row_p, col_p, normp, xw2, z2d),)
    if isinstance(part2, (tuple, list)):
        part2 = part2[0]
    out = _m3(part2[0], part2[1], xw2, dis2d, b2.reshape(1, D), W3,
              b3.reshape(1, D))
    return out[:N]


# speedup vs baseline: 8.9908x; 1.1364x over previous
import functools

import jax
import jax.numpy as jnp
from jax import lax
from jax.experimental import pallas as pl
from jax.experimental.pallas import tpu as pltpu
from jax.experimental.pallas import tpu_sc as plsc

N = 10000
NP = 10240
D = 128
DH = D // 2
E = 320000
NC = 2
NS = 16
K = 128
NBUF = 4
NCH = 160
NPASS = 2
PCH = NCH // NPASS
EPAD = NS * NCH * K
ECH = EPAD // K
RPT = NP // NS
L = 16
BM = 256
GB = NP // BM


def _rsqrt16(d):
    i = lax.bitcast_convert_type(d, jnp.int32)
    i = jnp.int32(0x5F3759DF) - (i >> 1)
    r = lax.bitcast_convert_type(i, jnp.float32)
    for _ in range(3):
        r = r * (1.5 - 0.5 * d * r * r)
    return r


def _bcast_lane(v, j):
    return jnp.take_along_axis(
        v, jnp.full((L,), j, jnp.int32), axis=0, mode="promise_in_bounds"
    )


def _pipeline_pass(xwcat_hbm, acc_sp, bufA, bufB, bufC, rows, sgs, sss,
                   dis_sp=None, dis_chunk=None):
    first = dis_sp is not None
    ss = sss[0]

    def g_start(i, b):
        pltpu.async_copy(xwcat_hbm.at[bufA.at[i]], rows[b], sgs[b])

    def g_wait(i, b):
        pltpu.make_async_copy(xwcat_hbm.at[bufA.at[i]], rows[b], sgs[b]).wait()

    def s_wait(i, b):
        pltpu.make_async_copy(rows[b], acc_sp.at[bufC.at[i]], ss).wait()

    for b in range(NBUF):
        g_start(b, b)

    @pl.loop(0, PCH // NBUF)
    def _blk(ii):
        for b in range(NBUF):
            i = ii * NBUF + b
            if first:
                pltpu.sync_copy(dis_sp.at[bufA.at[i]], dis_chunk)
            g_wait(i, b)

            @pl.loop(0, K // L)
            def _grp(g):
                g16 = pl.multiple_of(g * L, L)
                sl = pl.ds(g16, L)
                if first:
                    bufB[i, sl] = bufB[i, sl] * dis_chunk[sl]
                nv = bufB[i, sl]
                for j in range(L):
                    nb = _bcast_lane(nv, j)
                    e = g16 + j
                    for d in range(DH // L):
                        sld = pl.ds(d * L, L)
                        rows[b][e, sld] = rows[b][e, sld] * nb

            bp = (b - 1) % NBUF

            @pl.when(i >= 1)
            def _():
                s_wait(i - 1, bp)

                @pl.when(i + NBUF - 1 < PCH)
                def _():
                    g_start(i + NBUF - 1, bp)

            pltpu.async_copy(rows[b], acc_sp.at[bufC.at[i]], ss, add=True)

    s_wait(PCH - 1, (PCH - 1) % NBUF)


def _load_edges(row_hbm, col_hbm, w_hbm, bufA, bufB, bufC, s, c, p):
    base = s * NCH + p * PCH
    pltpu.sync_copy(row_hbm.at[pl.ds(base, PCH)], bufA)
    pltpu.sync_copy(w_hbm.at[pl.ds(base, PCH)], bufB)
    pltpu.sync_copy(col_hbm.at[pl.ds(base, PCH)], bufC)
    cnp = c * NP

    @pl.loop(0, PCH)
    def _ofs(i):
        for g in range(K // L):
            sl = pl.ds(g * L, L)
            bufA[i, sl] = bufA[i, sl] + cnp


def _sc1_body(row_hbm, col_hbm, ew_hbm, xwcat_hbm, z2d_hbm, z1d_hbm,
              acc_hbm, dis_hbm, normp_hbm,
              acc_sp, deg_sp, dis_sp,
              bufA, bufB, bufC, rows0, rows1, rows2, rows3, small, dis_chunk,
              sg0, sg1, sg2, sg3, ss0):
    c = lax.axis_index("c")
    s = lax.axis_index("s")
    rows = (rows0, rows1, rows2, rows3)
    sgs = (sg0, sg1, sg2, sg3)
    sss = (ss0,)

    pltpu.sync_copy(z2d_hbm, acc_sp.at[pl.ds(s * RPT, RPT)])
    pltpu.sync_copy(z1d_hbm, deg_sp.at[pl.ds(s * RPT, RPT)])
    plsc.subcore_barrier()

    @pl.loop(0, NPASS)
    def _dgp(p):
        base = s * NCH + p * PCH
        pltpu.sync_copy(ew_hbm.at[pl.ds(base, PCH)], bufB)
        pltpu.sync_copy(col_hbm.at[pl.ds(base, PCH)], bufC)

        @pl.loop(0, PCH)
        def _dgi(i):
            pltpu.sync_copy(bufB.at[i], deg_sp.at[bufC.at[i]], add=True)

    plsc.subcore_barrier()

    for piece in range(RPT // 128):
        pbase = s * RPT + piece * 128
        pltpu.sync_copy(deg_sp.at[pl.ds(pbase, 128)], small)

        @pl.loop(0, 128 // L)
        def _dis(g):
            g16 = pl.multiple_of(g * L, L)
            small[pl.ds(g16, L)] = _rsqrt16(small[pl.ds(g16, L)] + 1.0)

        pltpu.sync_copy(small, dis_sp.at[pl.ds(pbase, 128)])
        pltpu.sync_copy(small, dis_sp.at[pl.ds(NP + pbase, 128)])

        @pl.when(c == 0)
        def _():
            pltpu.sync_copy(small, dis_hbm.at[pl.ds(pbase, 128)])

    plsc.subcore_barrier()

    @pl.loop(0, NPASS)
    def _mp(p):
        _load_edges(row_hbm, col_hbm, ew_hbm, bufA, bufB, bufC, s, c, p)
        _pipeline_pass(xwcat_hbm, acc_sp, bufA, bufB, bufC, rows, sgs, sss,
                       dis_sp=dis_sp, dis_chunk=dis_chunk)

        @pl.when(c == 0)
        def _():
            pltpu.sync_copy(bufB, normp_hbm.at[pl.ds(s * NCH + p * PCH, PCH)])

    plsc.subcore_barrier()

    pltpu.sync_copy(acc_sp.at[pl.ds(s * RPT, RPT)],
                    acc_hbm.at[pl.ds(c * NP + s * RPT, RPT)])


def _sc2_body(row_hbm, col_hbm, np_hbm, xwcat_hbm, z2d_hbm,
              acc_hbm,
              acc_sp, bufA, bufB, bufC, rows0, rows1, rows2, rows3,
              sg0, sg1, sg2, sg3, ss0):
    c = lax.axis_index("c")
    s = lax.axis_index("s")
    rows = (rows0, rows1, rows2, rows3)
    sgs = (sg0, sg1, sg2, sg3)
    sss = (ss0,)

    pltpu.sync_copy(z2d_hbm, acc_sp.at[pl.ds(s * RPT, RPT)])
    plsc.subcore_barrier()

    @pl.loop(0, NPASS)
    def _mp(p):
        _load_edges(row_hbm, col_hbm, np_hbm, bufA, bufB, bufC, s, c, p)
        _pipeline_pass(xwcat_hbm, acc_sp, bufA, bufB, bufC, rows, sgs, sss)

    plsc.subcore_barrier()

    pltpu.sync_copy(acc_sp.at[pl.ds(s * RPT, RPT)],
                    acc_hbm.at[pl.ds(c * NP + s * RPT, RPT)])


@functools.lru_cache(maxsize=None)
def _sc1():
    mesh = plsc.VectorSubcoreMesh(core_axis_name="c", subcore_axis_name="s")
    return pl.kernel(
        _sc1_body,
        out_type=[
            jax.ShapeDtypeStruct((NC * NP, DH), jnp.float32),
            jax.ShapeDtypeStruct((NP,), jnp.float32),
            jax.ShapeDtypeStruct((ECH, K), jnp.float32),
        ],
        mesh=mesh,
        compiler_params=pltpu.CompilerParams(
            needs_layout_passes=False, use_tc_tiling_on_sc=False),
        scratch_types=[
            pltpu.VMEM_SHARED((NP, DH), jnp.float32),
            pltpu.VMEM_SHARED((NP,), jnp.float32),
            pltpu.VMEM_SHARED((NC * NP,), jnp.float32),
            pltpu.VMEM((PCH, K), jnp.int32),
            pltpu.VMEM((PCH, K), jnp.float32),
            pltpu.VMEM((PCH, K), jnp.int32),
            pltpu.VMEM((K, DH), jnp.float32),
            pltpu.VMEM((K, DH), jnp.float32),
            pltpu.VMEM((K, DH), jnp.float32),
            pltpu.VMEM((K, DH), jnp.float32),
            pltpu.VMEM((128,), jnp.float32),
            pltpu.VMEM((K,), jnp.float32),
        ] + [pltpu.SemaphoreType.DMA] * 5,
    )


@functools.lru_cache(maxsize=None)
def _sc2():
    mesh = plsc.VectorSubcoreMesh(core_axis_name="c", subcore_axis_name="s")
    return pl.kernel(
        _sc2_body,
        out_type=[jax.ShapeDtypeStruct((NC * NP, DH), jnp.float32)],
        mesh=mesh,
        compiler_params=pltpu.CompilerParams(
            needs_layout_passes=False, use_tc_tiling_on_sc=False),
        scratch_types=[
            pltpu.VMEM_SHARED((NP, DH), jnp.float32),
            pltpu.VMEM((PCH, K), jnp.int32),
            pltpu.VMEM((PCH, K), jnp.float32),
            pltpu.VMEM((PCH, K), jnp.int32),
            pltpu.VMEM((K, DH), jnp.float32),
            pltpu.VMEM((K, DH), jnp.float32),
            pltpu.VMEM((K, DH), jnp.float32),
            pltpu.VMEM((K, DH), jnp.float32),
        ] + [pltpu.SemaphoreType.DMA] * 5,
    )


def _m1_body(x_ref, w_ref, o_ref):
    o_ref[...] = jnp.dot(x_ref[...], w_ref[0],
                         preferred_element_type=jnp.float32)


def _m2_body(p_lo, p_hi, xw_lo, xw_hi, dis_ref, b_ref, w_ref, o_ref):
    d = dis_ref[...]
    pcat = jnp.concatenate([p_lo[...], p_hi[...]], axis=1)
    xcat = jnp.concatenate([xw_lo[...], xw_hi[...]], axis=1)
    h = jnp.maximum((pcat + d * xcat) * d + b_ref[...], 0.0)
    o_ref[...] = jnp.dot(h, w_ref[0], preferred_element_type=jnp.float32)


def _m3_body(p_lo, p_hi, xw_lo, xw_hi, dis_ref, b_ref, w_ref, b3_ref, o_ref):
    d = dis_ref[...]
    pcat = jnp.concatenate([p_lo[...], p_hi[...]], axis=1)
    xcat = jnp.concatenate([xw_lo[...], xw_hi[...]], axis=1)
    h = jnp.maximum((pcat + d * xcat) * d + b_ref[...], 0.0)
    o_ref[...] = jnp.tanh(
        jnp.dot(h, w_ref[...], preferred_element_type=jnp.float32)
        + b3_ref[...]
    )


_lo_spec = pl.BlockSpec((BM, DH), lambda i, c: (i, 0))
_hi_spec = pl.BlockSpec((BM, DH), lambda i, c: (GB + i, 0))
_whalf_spec = pl.BlockSpec((1, D, DH), lambda i, c: (c, 0, 0))
_ohalf_spec = pl.BlockSpec((BM, DH), lambda i, c: (c * GB + i, 0))
_dis_spec = pl.BlockSpec((BM, 1), lambda i, c: (i, 0))
_b_spec = pl.BlockSpec((1, D), lambda i, c: (0, 0))
_cat_sds = jax.ShapeDtypeStruct((NC * NP, DH), jnp.float32)


def _m1(x, w):
    return pl.pallas_call(
        _m1_body,
        grid=(GB, NC),
        in_specs=[pl.BlockSpec((BM, D), lambda i, c: (i, 0)), _whalf_spec],
        out_specs=_ohalf_spec,
        out_shape=_cat_sds,
    )(x, w)


def _m2(acc, xwcat, dis2d, b, w):
    return pl.pallas_call(
        _m2_body,
        grid=(GB, NC),
        in_specs=[_lo_spec, _hi_spec, _lo_spec, _hi_spec, _dis_spec, _b_spec,
                  _whalf_spec],
        out_specs=_ohalf_spec,
        out_shape=_cat_sds,
    )(acc, acc, xwcat, xwcat, dis2d, b, w)


def _m3(acc, xwcat, dis2d, b, w, b3):
    return pl.pallas_call(
        _m3_body,
        grid=(GB,),
        in_specs=[
            pl.BlockSpec((BM, DH), lambda i: (i, 0)),
            pl.BlockSpec((BM, DH), lambda i: (GB + i, 0)),
            pl.BlockSpec((BM, DH), lambda i: (i, 0)),
            pl.BlockSpec((BM, DH), lambda i: (GB + i, 0)),
            pl.BlockSpec((BM, 1), lambda i: (i, 0)),
            pl.BlockSpec((1, D), lambda i: (0, 0)),
            pl.BlockSpec((D, D), lambda i: (0, 0)),
            pl.BlockSpec((1, D), lambda i: (0, 0)),
        ],
        out_specs=pl.BlockSpec((BM, D), lambda i: (i, 0)),
        out_shape=jax.ShapeDtypeStruct((NP, D), jnp.float32),
    )(acc, acc, xwcat, xwcat, dis2d, b, w, b3)


def kernel(x, edge_index, edge_weight, W1, b1, W2, b2, W3, b3):
    row = edge_index[0].astype(jnp.int32)
    col = edge_index[1].astype(jnp.int32)
    ew = edge_weight.astype(jnp.float32)

    pad = EPAD - E
    row_p = jnp.concatenate([row, jnp.zeros((pad,), jnp.int32)]).reshape(ECH, K)
    col_p = jnp.concatenate([col, jnp.full((pad,), N, jnp.int32)]).reshape(ECH, K)
    ew_p = jnp.concatenate([ew, jnp.zeros((pad,), jnp.float32)]).reshape(ECH, K)
    x_p = jnp.pad(x, ((0, NP - N), (0, 0)))
    z2d = jnp.zeros((RPT, DH), jnp.float32)
    z1d = jnp.zeros((RPT,), jnp.float32)

    w1h = W1.astype(jnp.float32).reshape(D, NC, DH).transpose(1, 0, 2)
    w2h = W2.astype(jnp.float32).reshape(D, NC, DH).transpose(1, 0, 2)
    xw1 = _m1(x_p, w1h)
    acc1, dis, normp = _sc1()(row_p, col_p, ew_p, xw1, z2d, z1d)
    dis2d = dis[:, None]
    xw2 = _m2(acc1, xw1, dis2d, b1.reshape(1, D), w2h)
    acc2 = _sc2()(row_p, col_p, normp, xw2, z2d)
    if isinstance(acc2, (tuple, list)):
        acc2 = acc2[0]
    out = _m3(acc2, xw2, dis2d, b2.reshape(1, D), W3, b3.reshape(1, D))
    return out[:N]


# speedup vs baseline: 9.7700x; 1.0867x over previous
import functools

import jax
import jax.numpy as jnp
from jax import lax
from jax.experimental import pallas as pl
from jax.experimental.pallas import tpu as pltpu
from jax.experimental.pallas import tpu_sc as plsc

N = 10000
NP = 10240
D = 128
DH = D // 2
E = 320000
NC = 2
NS = 16
NW = NC * NS
K = 128
NBUF = 4
NCH = 160
NPASS = 2
PCH = NCH // NPASS
EPAD = NS * NCH * K
ECH = EPAD // K
RPT = NP // NS
L = 16
BM = 256
GB = NP // BM


def _bcast_lane(v, j):
    return jnp.take_along_axis(
        v, jnp.full((L,), j, jnp.int32), axis=0, mode="promise_in_bounds"
    )


def _sc_deg_body(col_hbm, ew_hbm, z1d_hbm,
                 deg2_hbm,
                 deg_sp, bufB, bufC):
    c = lax.axis_index("c")
    s = lax.axis_index("s")
    w = s * NC + c

    pltpu.sync_copy(z1d_hbm, deg_sp.at[pl.ds(s * RPT, RPT)])
    plsc.subcore_barrier()

    base = w * PCH
    pltpu.sync_copy(ew_hbm.at[pl.ds(base, PCH)], bufB)
    pltpu.sync_copy(col_hbm.at[pl.ds(base, PCH)], bufC)

    @pl.loop(0, PCH)
    def _dgi(i):
        pltpu.sync_copy(bufB.at[i], deg_sp.at[bufC.at[i]], add=True)

    plsc.subcore_barrier()
    pltpu.sync_copy(deg_sp.at[pl.ds(s * RPT, RPT)],
                    deg2_hbm.at[c, pl.ds(s * RPT, RPT)])


@functools.lru_cache(maxsize=None)
def _sc_deg():
    mesh = plsc.VectorSubcoreMesh(core_axis_name="c", subcore_axis_name="s")
    return pl.kernel(
        _sc_deg_body,
        out_type=[jax.ShapeDtypeStruct((NC, NP), jnp.float32)],
        mesh=mesh,
        compiler_params=pltpu.CompilerParams(
            needs_layout_passes=False, use_tc_tiling_on_sc=False),
        scratch_types=[
            pltpu.VMEM_SHARED((NP,), jnp.float32),
            pltpu.VMEM((PCH, K), jnp.float32),
            pltpu.VMEM((PCH, K), jnp.int32),
        ],
    )


def _pipeline_pass(ycat_hbm, acc_sp, bufA, bufB, bufC, rows, sgs, ss):
    def g_start(i, b):
        pltpu.async_copy(ycat_hbm.at[bufA.at[i]], rows[b], sgs[b])

    def g_wait(i, b):
        pltpu.make_async_copy(ycat_hbm.at[bufA.at[i]], rows[b], sgs[b]).wait()

    def s_wait(i, b):
        pltpu.make_async_copy(rows[b], acc_sp.at[bufC.at[i]], ss).wait()

    for b in range(NBUF):
        g_start(b, b)

    @pl.loop(0, PCH // NBUF)
    def _blk(ii):
        for b in range(NBUF):
            i = ii * NBUF + b
            g_wait(i, b)

            @pl.loop(0, K // L)
            def _grp(g):
                g16 = pl.multiple_of(g * L, L)
                nv = bufB[i, pl.ds(g16, L)]
                for j in range(L):
                    nb = _bcast_lane(nv, j)
                    e = g16 + j
                    for d in range(DH // L):
                        sld = pl.ds(d * L, L)
                        rows[b][e, sld] = rows[b][e, sld] * nb

            bp = (b - 1) % NBUF

            @pl.when(i >= 1)
            def _():
                s_wait(i - 1, bp)

                @pl.when(i + NBUF - 1 < PCH)
                def _():
                    g_start(i + NBUF - 1, bp)

            pltpu.async_copy(rows[b], acc_sp.at[bufC.at[i]], ss, add=True)

    s_wait(PCH - 1, (PCH - 1) % NBUF)


def _sc_layer_body(row_hbm, col_hbm, ew_hbm, ycat_hbm, z2d_hbm,
                   acc_hbm,
                   acc_sp, bufA, bufB, bufC, rows0, rows1, rows2, rows3,
                   sg0, sg1, sg2, sg3, ss0):
    c = lax.axis_index("c")
    s = lax.axis_index("s")
    rows = (rows0, rows1, rows2, rows3)
    sgs = (sg0, sg1, sg2, sg3)

    pltpu.sync_copy(z2d_hbm, acc_sp.at[pl.ds(s * RPT, RPT)])
    plsc.subcore_barrier()

    cnp = c * NP

    @pl.loop(0, NPASS)
    def _mp(p):
        base = s * NCH + p * PCH
        pltpu.sync_copy(row_hbm.at[pl.ds(base, PCH)], bufA)
        pltpu.sync_copy(ew_hbm.at[pl.ds(base, PCH)], bufB)
        pltpu.sync_copy(col_hbm.at[pl.ds(base, PCH)], bufC)

        @pl.loop(0, PCH)
        def _ofs(i):
            for g in range(K // L):
                sl = pl.ds(g * L, L)
                bufA[i, sl] = bufA[i, sl] + cnp

        _pipeline_pass(ycat_hbm, acc_sp, bufA, bufB, bufC, rows, sgs, ss0)

    plsc.subcore_barrier()
    pltpu.sync_copy(acc_sp.at[pl.ds(s * RPT, RPT)],
                    acc_hbm.at[pl.ds(c * NP + s * RPT, RPT)])


@functools.lru_cache(maxsize=None)
def _sc_layer():
    mesh = plsc.VectorSubcoreMesh(core_axis_name="c", subcore_axis_name="s")
    return pl.kernel(
        _sc_layer_body,
        out_type=[jax.ShapeDtypeStruct((NC * NP, DH), jnp.float32)],
        mesh=mesh,
        compiler_params=pltpu.CompilerParams(
            needs_layout_passes=False, use_tc_tiling_on_sc=False),
        scratch_types=[
            pltpu.VMEM_SHARED((NP, DH), jnp.float32),
            pltpu.VMEM((PCH, K), jnp.int32),
            pltpu.VMEM((PCH, K), jnp.float32),
            pltpu.VMEM((PCH, K), jnp.int32),
            pltpu.VMEM((K, DH), jnp.float32),
            pltpu.VMEM((K, DH), jnp.float32),
            pltpu.VMEM((K, DH), jnp.float32),
            pltpu.VMEM((K, DH), jnp.float32),
        ] + [pltpu.SemaphoreType.DMA] * 5,
    )


def _dis_of(deg_lo_ref, deg_hi_ref):
    return lax.rsqrt(deg_lo_ref[...] + deg_hi_ref[...] + 1.0)


def _m1_body(x_ref, w_ref, dlo_ref, dhi_ref, o_ref):
    d = _dis_of(dlo_ref, dhi_ref)
    o_ref[...] = d * jnp.dot(x_ref[...], w_ref[0],
                             preferred_element_type=jnp.float32)


def _m2_body(p_lo, p_hi, y_lo, y_hi, dlo_ref, dhi_ref, b_ref, w_ref, o_ref):
    d = _dis_of(dlo_ref, dhi_ref)
    scat = jnp.concatenate(
        [p_lo[...] + y_lo[...], p_hi[...] + y_hi[...]], axis=1)
    h = jnp.maximum(d * scat + b_ref[...], 0.0)
    o_ref[...] = d * jnp.dot(h, w_ref[0], preferred_element_type=jnp.float32)


def _m3_body(p_lo, p_hi, y_lo, y_hi, dlo_ref, dhi_ref, b_ref, w_ref, b3_ref,
             o_ref):
    d = _dis_of(dlo_ref, dhi_ref)
    scat = jnp.concatenate(
        [p_lo[...] + y_lo[...], p_hi[...] + y_hi[...]], axis=1)
    h = jnp.maximum(d * scat + b_ref[...], 0.0)
    o_ref[...] = jnp.tanh(
        jnp.dot(h, w_ref[...], preferred_element_type=jnp.float32)
        + b3_ref[...]
    )


_lo_spec = pl.BlockSpec((BM, DH), lambda i, c: (i, 0))
_hi_spec = pl.BlockSpec((BM, DH), lambda i, c: (GB + i, 0))
_whalf_spec = pl.BlockSpec((1, D, DH), lambda i, c: (c, 0, 0))
_ohalf_spec = pl.BlockSpec((BM, DH), lambda i, c: (c * GB + i, 0))
_dis_spec = pl.BlockSpec((BM, 1), lambda i, c: (i, 0))
_b_spec = pl.BlockSpec((1, D), lambda i, c: (0, 0))
_cat_sds = jax.ShapeDtypeStruct((NC * NP, DH), jnp.float32)


def _m1(x, w, dlo, dhi):
    return pl.pallas_call(
        _m1_body,
        grid=(GB, NC),
        in_specs=[pl.BlockSpec((BM, D), lambda i, c: (i, 0)), _whalf_spec,
                  _dis_spec, _dis_spec],
        out_specs=_ohalf_spec,
        out_shape=_cat_sds,
    )(x, w, dlo, dhi)


def _m2(acc, ycat, dlo, dhi, b, w):
    return pl.pallas_call(
        _m2_body,
        grid=(GB, NC),
        in_specs=[_lo_spec, _hi_spec, _lo_spec, _hi_spec, _dis_spec,
                  _dis_spec, _b_spec, _whalf_spec],
        out_specs=_ohalf_spec,
        out_shape=_cat_sds,
    )(acc, acc, ycat, ycat, dlo, dhi, b, w)


def _m3(acc, ycat, dlo, dhi, b, w, b3):
    return pl.pallas_call(
        _m3_body,
        grid=(GB,),
        in_specs=[
            pl.BlockSpec((BM, DH), lambda i: (i, 0)),
            pl.BlockSpec((BM, DH), lambda i: (GB + i, 0)),
            pl.BlockSpec((BM, DH), lambda i: (i, 0)),
            pl.BlockSpec((BM, DH), lambda i: (GB + i, 0)),
            pl.BlockSpec((BM, 1), lambda i: (i, 0)),
            pl.BlockSpec((BM, 1), lambda i: (i, 0)),
            pl.BlockSpec((1, D), lambda i: (0, 0)),
            pl.BlockSpec((D, D), lambda i: (0, 0)),
            pl.BlockSpec((1, D), lambda i: (0, 0)),
        ],
        out_specs=pl.BlockSpec((BM, D), lambda i: (i, 0)),
        out_shape=jax.ShapeDtypeStruct((NP, D), jnp.float32),
    )(acc, acc, ycat, ycat, dlo, dhi, b, w, b3)


def _first(x):
    return x[0] if isinstance(x, (tuple, list)) else x


def kernel(x, edge_index, edge_weight, W1, b1, W2, b2, W3, b3):
    row = edge_index[0].astype(jnp.int32)
    col = edge_index[1].astype(jnp.int32)
    ew = edge_weight.astype(jnp.float32)

    pad = EPAD - E
    row_p = jnp.concatenate([row, jnp.zeros((pad,), jnp.int32)]).reshape(ECH, K)
    col_p = jnp.concatenate([col, jnp.full((pad,), N, jnp.int32)]).reshape(ECH, K)
    ew_p = jnp.concatenate([ew, jnp.zeros((pad,), jnp.float32)]).reshape(ECH, K)
    x_p = jnp.pad(x, ((0, NP - N), (0, 0)))
    z2d = jnp.zeros((RPT, DH), jnp.float32)
    z1d = jnp.zeros((RPT,), jnp.float32)
    w1h = W1.astype(jnp.float32).reshape(D, NC, DH).transpose(1, 0, 2)
    w2h = W2.astype(jnp.float32).reshape(D, NC, DH).transpose(1, 0, 2)

    deg2 = _first(_sc_deg()(col_p, ew_p, z1d))
    dlo = deg2[0][:, None]
    dhi = deg2[1][:, None]

    y1 = _m1(x_p, w1h, dlo, dhi)
    acc1 = _first(_sc_layer()(row_p, col_p, ew_p, y1, z2d))
    y2 = _m2(acc1, y1, dlo, dhi, b1.reshape(1, D), w2h)
    acc2 = _first(_sc_layer()(row_p, col_p, ew_p, y2, z2d))
    out = _m3(acc2, y2, dlo, dhi, b2.reshape(1, D), W3, b3.reshape(1, D))
    return out[:N]
